# baseline (speedup 1.0000x reference)
import jax
import jax.numpy as jnp
from jax import lax
from jax.experimental import pallas as pl
from jax.experimental.pallas import tpu as pltpu
from jax.experimental.pallas import tpu_sc as plsc

NUM_HEADS = 16
MAX_DIST = 128
S = 2048
NVAR = 16
VLEN = 4096
LANES = 16
CPAD = 304
RAMP0 = 1904
RIGHT0 = 2144


def _rpb_body(cols_hbm, out_hbm, col_v, vref, sem):
    c = lax.axis_index("c")
    s = lax.axis_index("s")
    h = s

    pltpu.sync_copy(cols_hbm.at[h], col_v)

    left = col_v[pl.ds(0, LANES)]
    right = col_v[pl.ds(CPAD - LANES, LANES)]

    for r in range(NVAR):
        base = r * VLEN

        def fill_left(i, carry, base=base):
            vref[pl.ds(base + i * LANES, LANES)] = left
            return carry

        def fill_right(i, carry, base=base):
            vref[pl.ds(base + RIGHT0 + i * LANES, LANES)] = right
            return carry

        def fill_ramp(i, carry, base=base, r=r):
            vals = col_v[pl.ds(LANES - r + i * LANES, LANES)]
            vref[pl.ds(base + RAMP0 + i * LANES, LANES)] = vals
            return carry

        lax.fori_loop(0, RAMP0 // LANES + 1, fill_left, 0)
        lax.fori_loop(0, (VLEN - RIGHT0) // LANES, fill_right, 0)
        lax.fori_loop(0, 18, fill_ramp, 0)

    g_base = c * 64
    LOOKAHEAD = 4

    def issue_chunk(j):
        q0 = (g_base + j) * NVAR
        b0 = 2032 - q0
        for r in range(NVAR):
            src_off = pl.multiple_of(r * VLEN + b0, 16)
            dst_off = pl.multiple_of((h * S + q0 + r) * S, S)
            pltpu.async_copy(
                vref.at[pl.ds(src_off, S)],
                out_hbm.at[pl.ds(dst_off, S)],
                sem,
            )

    def drain_row():
        pltpu.make_async_copy(
            out_hbm.at[pl.ds(0, S)], vref.at[pl.ds(0, S)], sem
        ).wait()

    for j in range(LOOKAHEAD):
        issue_chunk(j)

    def dma_body(j, carry):
        issue_chunk(j)
        for _ in range(NVAR):
            drain_row()
        return carry

    lax.fori_loop(LOOKAHEAD, 64, dma_body, 0)
    for _ in range(LOOKAHEAD * NVAR):
        drain_row()


def kernel(seq_length, table):
    del seq_length
    cols = jnp.pad(table.T, ((0, 0), (LANES, CPAD - LANES - (2 * MAX_DIST + 1))),
                   mode="edge")
    mesh = plsc.VectorSubcoreMesh(core_axis_name="c", subcore_axis_name="s")
    f = pl.kernel(
        _rpb_body,
        mesh=mesh,
        out_type=jax.ShapeDtypeStruct((NUM_HEADS * S * S,), jnp.float32),
        scratch_types=[
            pltpu.VMEM((CPAD,), jnp.float32),
            pltpu.VMEM((NVAR * VLEN,), jnp.float32),
            pltpu.SemaphoreType.DMA,
        ],
    )
    out = f(cols)
    return out.reshape(1, NUM_HEADS, S, S)


# speedup vs baseline: 3.3631x; 3.3631x over previous
import jax
import jax.numpy as jnp
from jax import lax
from jax.experimental import pallas as pl
from jax.experimental.pallas import tpu as pltpu
from jax.experimental.pallas import tpu_sc as plsc

NUM_HEADS = 16
MAX_DIST = 128
S = 2048
LANES = 16
NCLS = 50
CPAD = 576
ROWT = S // 8
COLT = S // 128
INFLIGHT = 32


def _rpb_body(cols_hbm, out_hbm, col_v, tiles_v, sem):
    c = lax.axis_index("c")
    s = lax.axis_index("s")
    h = s

    pltpu.sync_copy(cols_hbm.at[pl.ds(h * CPAD, CPAD)], col_v)

    def build_body(n, carry):
        cls = n // 64 - 32
        i = (n // 8) % 8
        jj = n % 8
        vals = col_v[pl.ds(288 + 8 * cls - i + jj * LANES, LANES)]
        tiles_v[n // 64, i, pl.ds(jj * LANES, LANES)] = vals
        return carry

    lax.fori_loop(0, NCLS * 64, build_body, 0)

    a_base = c * (ROWT // 2)
    n_tiles = (ROWT // 2) * COLT

    def issue(n):
        b = n // (ROWT // 2)
        a = a_base + n % (ROWT // 2)
        cls_idx = jnp.clip(16 * b - a, -32, 17) + 32
        pltpu.async_copy(
            tiles_v.at[cls_idx],
            out_hbm.at[0, h,
                       pl.ds(pl.multiple_of(a * 8, 8), 8),
                       pl.ds(pl.multiple_of(b * 128, 128), 128)],
            sem,
        )

    def drain():
        pltpu.make_async_copy(
            out_hbm.at[0, 0, pl.ds(0, 8), pl.ds(0, 128)], tiles_v.at[0], sem
        ).wait()

    def prime_body(n, carry):
        issue(n)
        return carry

    def steady_body(n, carry):
        issue(n)
        drain()
        return carry

    def tail_body(n, carry):
        drain()
        return carry

    lax.fori_loop(0, INFLIGHT, prime_body, 0)
    lax.fori_loop(INFLIGHT, n_tiles, steady_body, 0)
    lax.fori_loop(0, INFLIGHT, tail_body, 0)


def kernel(seq_length, table):
    del seq_length
    cols = jnp.pad(table.T, ((0, 0), (160, CPAD - 160 - (2 * MAX_DIST + 1))),
                   mode="edge").reshape(-1)
    mesh = plsc.VectorSubcoreMesh(core_axis_name="c", subcore_axis_name="s")
    f = pl.kernel(
        _rpb_body,
        mesh=mesh,
        out_type=jax.ShapeDtypeStruct((1, NUM_HEADS, S, S), jnp.float32),
        scratch_types=[
            pltpu.VMEM((CPAD,), jnp.float32),
            pltpu.VMEM((NCLS, 8, 128), jnp.float32),
            pltpu.SemaphoreType.DMA,
        ],
        compiler_params=pltpu.CompilerParams(use_tc_tiling_on_sc=True),
    )
    return f(cols)
